# Initial kernel scaffold; baseline (speedup 1.0000x reference)
import jax
import jax.numpy as jnp
from jax.experimental import pallas as pl


def kernel(x, w1, v1, w2, w_router):
    raise NotImplementedError("write your pallas kernel here")


# speedup vs baseline: 1.2263x; 1.2263x over previous
import functools

import jax
import jax.numpy as jnp
from jax.experimental import pallas as pl
from jax.experimental.pallas import tpu as pltpu

S = 2048
D = 768
FFN = 3072
E = 8
K = 2
F_TILE = 256
NF = FFN // F_TILE


def _router_body(x_ref, wr_ref, weights_ref, gate_ref):
    x = x_ref[...]
    logits = jax.lax.dot_general(
        x, wr_ref[...], (((1,), (0,)), ((), ())),
        preferred_element_type=jnp.float32,
    )
    m = jnp.max(logits, axis=-1, keepdims=True)
    ex = jnp.exp(logits - m)
    weights = ex / jnp.sum(ex, axis=-1, keepdims=True)
    weights_ref[...] = weights

    eidx = jax.lax.broadcasted_iota(jnp.int32, (S, E), 1)
    m1 = jnp.max(weights, axis=-1, keepdims=True)
    is1 = weights >= m1
    a1 = jnp.min(jnp.where(is1, eidx, E), axis=-1, keepdims=True)
    w_excl = jnp.where(eidx == a1, -jnp.inf, weights)
    m2 = jnp.max(w_excl, axis=-1, keepdims=True)
    is2 = w_excl >= m2
    a2 = jnp.min(jnp.where(is2, eidx, E), axis=-1, keepdims=True)
    gate = jnp.where(eidx == a1, m1, 0.0) + jnp.where(eidx == a2, m2, 0.0)
    gate_ref[...] = gate.T.reshape(E, 1, S)


def _ffn_body(gate_ref, x_ref, w1_ref, v1_ref, w2_ref, out_ref, acc_ref):
    e = pl.program_id(0)
    f = pl.program_id(1)
    x = x_ref[...]
    w1 = w1_ref[0]
    v1 = v1_ref[0]
    w2 = w2_ref[0]
    x1 = jax.lax.dot_general(x, w1, (((1,), (1,)), ((), ())),
                             preferred_element_type=jnp.float32)
    x2 = jax.lax.dot_general(x, v1, (((1,), (1,)), ((), ())),
                             preferred_element_type=jnp.float32)
    h = x1 * jax.lax.logistic(x1) * x2
    p = jax.lax.dot_general(h, w2, (((1,), (0,)), ((), ())),
                            preferred_element_type=jnp.float32)
    g = gate_ref[0, 0, :].reshape(S, 1)
    p = g * p

    @pl.when(jnp.logical_and(e == 0, f == 0))
    def _():
        acc_ref[...] = p

    @pl.when(jnp.logical_not(jnp.logical_and(e == 0, f == 0)))
    def _():
        acc_ref[...] += p

    @pl.when(jnp.logical_and(e == E - 1, f == NF - 1))
    def _():
        out_ref[...] = acc_ref[...]


def kernel(x, w1, v1, w2, w_router):
    x2d = x.reshape(S, D)
    weights, gate = pl.pallas_call(
        _router_body,
        out_shape=(
            jax.ShapeDtypeStruct((S, E), jnp.float32),
            jax.ShapeDtypeStruct((E, 1, S), jnp.float32),
        ),
    )(x2d, w_router)

    ew1 = w1.reshape(E, FFN, D)
    ev1 = v1.reshape(E, FFN, D)
    ew2 = w2.reshape(E, FFN, D)

    wspec = pl.BlockSpec((1, F_TILE, D), lambda e, f: (e, f, 0))
    y = pl.pallas_call(
        _ffn_body,
        grid=(E, NF),
        in_specs=[
            pl.BlockSpec((1, 1, S), lambda e, f: (e, 0, 0)),
            pl.BlockSpec((S, D), lambda e, f: (0, 0)),
            wspec, wspec, wspec,
        ],
        out_specs=pl.BlockSpec((S, D), lambda e, f: (0, 0)),
        out_shape=jax.ShapeDtypeStruct((S, D), jnp.float32),
        scratch_shapes=[pltpu.VMEM((S, D), jnp.float32)],
        compiler_params=pltpu.CompilerParams(
            dimension_semantics=("arbitrary", "arbitrary"),
        ),
    )(gate, x2d, ew1, ev1, ew2)

    return y.reshape(1, S, D), weights.reshape(1, S, E)
